# baseline (speedup 1.0000x reference)
import jax
import jax.numpy as jnp
from jax import lax
from jax.experimental import pallas as pl
from jax.experimental.pallas import tpu as pltpu
from jax.experimental.pallas import tpu_sc as plsc

_NUM_VIEWS = 8
_PROMPT_LEN = 50
_DIM = 768
_BATCH = 1024
_ROW = _PROMPT_LEN * _DIM

_info = plsc.get_sparse_core_info()
_NC, _NS = _info.num_cores, _info.num_subcores
_NW = _NC * _NS
_BPW = _BATCH // _NW
_CHUNK = 2


def _sc_gather(view_id, table):
    mesh = plsc.VectorSubcoreMesh(core_axis_name="c", subcore_axis_name="s")

    @pl.kernel(
        mesh=mesh,
        out_type=jax.ShapeDtypeStruct((_BATCH, _ROW), jnp.float32),
        scratch_types=[
            pltpu.VMEM((_BPW // _CHUNK, _CHUNK), jnp.int32),
            pltpu.VMEM((_CHUNK, _ROW), jnp.float32),
            pltpu.SemaphoreType.DMA,
        ],
    )
    def k(idx_hbm, table_hbm, out_hbm, idx_v, buf, sem):
        wid = lax.axis_index("s") * _NC + lax.axis_index("c")
        base = wid * _BPW
        pltpu.sync_copy(idx_hbm.at[pl.ds(wid * (_BPW // _CHUNK), _BPW // _CHUNK)], idx_v)
        for j in range(_BPW // _CHUNK):
            pltpu.async_copy(table_hbm.at[idx_v.at[j]], buf, sem).wait()
            pltpu.sync_copy(buf, out_hbm.at[pl.ds(base + j * _CHUNK, _CHUNK)])

    return k(view_id.reshape(_BATCH // _CHUNK, _CHUNK), table)


def kernel(view_id, prompts):
    table = prompts.reshape(_NUM_VIEWS, _ROW)
    out = _sc_gather(view_id.astype(jnp.int32), table)
    return out.reshape(_BATCH, _PROMPT_LEN, _DIM)


# speedup vs baseline: 1.0097x; 1.0097x over previous
import jax
import jax.numpy as jnp
from jax import lax
from jax.experimental import pallas as pl
from jax.experimental.pallas import tpu as pltpu
from jax.experimental.pallas import tpu_sc as plsc

_NUM_VIEWS = 8
_PROMPT_LEN = 50
_DIM = 768
_BATCH = 1024
_ROW = _PROMPT_LEN * _DIM

_info = plsc.get_sparse_core_info()
_NC, _NS = _info.num_cores, _info.num_subcores
_NW = _NC * _NS
_BPW = _BATCH // _NW
_NBUF = 2


def _sc_gather(view_id, table):
    mesh = plsc.VectorSubcoreMesh(core_axis_name="c", subcore_axis_name="s")

    @pl.kernel(
        mesh=mesh,
        out_type=jax.ShapeDtypeStruct((_BATCH, _ROW), jnp.float32),
        scratch_types=[
            pltpu.VMEM((_BPW, 1), jnp.int32),
            pltpu.VMEM((1, _ROW), jnp.float32),
            pltpu.VMEM((1, _ROW), jnp.float32),
            pltpu.SemaphoreType.DMA,
            pltpu.SemaphoreType.DMA,
            pltpu.SemaphoreType.DMA,
            pltpu.SemaphoreType.DMA,
        ],
    )
    def k(idx_hbm, table_hbm, out_hbm, idx_v, buf0, buf1, g0, g1, s0, s1):
        wid = lax.axis_index("s") * _NC + lax.axis_index("c")
        base = wid * _BPW
        bufs, gsems, ssems = (buf0, buf1), (g0, g1), (s0, s1)
        pltpu.sync_copy(idx_hbm.at[pl.ds(base, _BPW)], idx_v)
        for b in range(_NBUF):
            pltpu.make_async_copy(
                table_hbm.at[idx_v.at[b]], bufs[b], gsems[b]
            ).start()

        def body(i, _):
            for b in range(_NBUF):
                j = _NBUF * i + b
                pltpu.make_async_copy(
                    table_hbm.at[pl.ds(0, 1)], bufs[b], gsems[b]
                ).wait()
                pltpu.make_async_copy(
                    bufs[b], out_hbm.at[pl.ds(base + j, 1)], ssems[b]
                ).start()
            for b in range(_NBUF):
                j = _NBUF * i + b
                pltpu.make_async_copy(
                    bufs[b], out_hbm.at[pl.ds(base, 1)], ssems[b]
                ).wait()

                @pl.when(i < _BPW // _NBUF - 1)
                def _():
                    pltpu.make_async_copy(
                        table_hbm.at[idx_v.at[j + _NBUF]], bufs[b], gsems[b]
                    ).start()

            return ()

        lax.fori_loop(0, _BPW // _NBUF, body, (), unroll=False)

    return k(view_id.reshape(_BATCH, 1), table)


def kernel(view_id, prompts):
    table = prompts.reshape(_NUM_VIEWS, _ROW)
    out = _sc_gather(view_id.astype(jnp.int32), table)
    return out.reshape(_BATCH, _PROMPT_LEN, _DIM)


# speedup vs baseline: 1.3885x; 1.3752x over previous
import jax
import jax.numpy as jnp
from jax import lax
from jax.experimental import pallas as pl
from jax.experimental.pallas import tpu as pltpu
from jax.experimental.pallas import tpu_sc as plsc

_NUM_VIEWS = 8
_PROMPT_LEN = 50
_DIM = 768
_BATCH = 1024
_PLEN_PAD = 56

_info = plsc.get_sparse_core_info()
_NC, _NS = _info.num_cores, _info.num_subcores
_NW = _NC * _NS
_BPW = _BATCH // _NW
_NBUF = 2


def _sc_gather(view_id, table):
    mesh = plsc.VectorSubcoreMesh(core_axis_name="c", subcore_axis_name="s")

    @pl.kernel(
        mesh=mesh,
        out_type=jax.ShapeDtypeStruct((_BATCH, _PLEN_PAD, _DIM), jnp.float32),
        scratch_types=[
            pltpu.VMEM((_BPW, 1), jnp.int32),
            pltpu.VMEM((1, _PLEN_PAD, _DIM), jnp.float32),
            pltpu.VMEM((1, _PLEN_PAD, _DIM), jnp.float32),
            pltpu.SemaphoreType.DMA,
            pltpu.SemaphoreType.DMA,
            pltpu.SemaphoreType.DMA,
            pltpu.SemaphoreType.DMA,
        ],
    )
    def k(idx_hbm, table_hbm, out_hbm, idx_v, buf0, buf1, g0, g1, s0, s1):
        wid = lax.axis_index("s") * _NC + lax.axis_index("c")
        base = wid * _BPW
        bufs, gsems, ssems = (buf0, buf1), (g0, g1), (s0, s1)
        pltpu.sync_copy(idx_hbm.at[pl.ds(base, _BPW)], idx_v)
        for b in range(_NBUF):
            pltpu.make_async_copy(
                table_hbm.at[idx_v.at[b]], bufs[b], gsems[b]
            ).start()

        def body(i, _):
            for b in range(_NBUF):
                j = _NBUF * i + b
                pltpu.make_async_copy(
                    table_hbm.at[pl.ds(0, 1)], bufs[b], gsems[b]
                ).wait()
                pltpu.make_async_copy(
                    bufs[b], out_hbm.at[pl.ds(base + j, 1)], ssems[b]
                ).start()
            for b in range(_NBUF):
                j = _NBUF * i + b
                pltpu.make_async_copy(
                    bufs[b], out_hbm.at[pl.ds(base, 1)], ssems[b]
                ).wait()

                @pl.when(i < _BPW // _NBUF - 1)
                def _():
                    pltpu.make_async_copy(
                        table_hbm.at[idx_v.at[j + _NBUF]], bufs[b], gsems[b]
                    ).start()

            return ()

        lax.fori_loop(0, _BPW // _NBUF, body, (), unroll=False)

    return k(view_id.reshape(_BATCH, 1), table)


def kernel(view_id, prompts):
    table = jnp.pad(prompts, ((0, 0), (0, _PLEN_PAD - _PROMPT_LEN), (0, 0)))
    out = _sc_gather(view_id.astype(jnp.int32), table)
    return out[:, :_PROMPT_LEN, :]


# speedup vs baseline: 2.1756x; 1.5668x over previous
import jax
import jax.numpy as jnp
from jax.experimental import pallas as pl
from jax.experimental.pallas import tpu as pltpu

_NUM_VIEWS = 8
_PROMPT_LEN = 50
_DIM = 768
_BATCH = 1024
_BB = 8


def _tc_gather(view_id, prompts):
    def body(idx_ref, tbl_ref, out_ref):
        i = pl.program_id(0)
        for r in range(_BB):
            v = idx_ref[i * _BB + r]
            out_ref[r] = tbl_ref[v]

    return pl.pallas_call(
        body,
        grid_spec=pltpu.PrefetchScalarGridSpec(
            num_scalar_prefetch=1,
            grid=(_BATCH // _BB,),
            in_specs=[
                pl.BlockSpec((_NUM_VIEWS, _PROMPT_LEN, _DIM), lambda i, idx: (0, 0, 0)),
            ],
            out_specs=pl.BlockSpec((_BB, _PROMPT_LEN, _DIM), lambda i, idx: (i, 0, 0)),
        ),
        out_shape=jax.ShapeDtypeStruct((_BATCH, _PROMPT_LEN, _DIM), jnp.float32),
    )(view_id, prompts)


def kernel(view_id, prompts):
    return _tc_gather(view_id.astype(jnp.int32), prompts)


# speedup vs baseline: 2.4819x; 1.1408x over previous
import jax
import jax.numpy as jnp
from jax.experimental import pallas as pl
from jax.experimental.pallas import tpu as pltpu

_NUM_VIEWS = 8
_PROMPT_LEN = 50
_DIM = 768
_BATCH = 1024
_BB = 32


def _tc_gather(view_id, prompts):
    def body(idx_ref, tbl_ref, out_ref):
        i = pl.program_id(0)
        for r in range(_BB):
            v = idx_ref[i * _BB + r]
            out_ref[r] = tbl_ref[v]

    return pl.pallas_call(
        body,
        grid_spec=pltpu.PrefetchScalarGridSpec(
            num_scalar_prefetch=1,
            grid=(_BATCH // _BB,),
            in_specs=[
                pl.BlockSpec((_NUM_VIEWS, _PROMPT_LEN, _DIM), lambda i, idx: (0, 0, 0)),
            ],
            out_specs=pl.BlockSpec((_BB, _PROMPT_LEN, _DIM), lambda i, idx: (i, 0, 0)),
        ),
        out_shape=jax.ShapeDtypeStruct((_BATCH, _PROMPT_LEN, _DIM), jnp.float32),
    )(view_id, prompts)


def kernel(view_id, prompts):
    return _tc_gather(view_id.astype(jnp.int32), prompts)
